# baseline (speedup 1.0000x reference)
import jax
import jax.numpy as jnp
from jax import lax
from jax.experimental import pallas as pl
from jax.experimental.pallas import tpu as pltpu
from jax.experimental.pallas import tpu_sc as plsc

VOCAB = 1000000
DIM = 64
BATCH = 4096
HIST = 200

NC = 2
NS = 16
NW = NC * NS
LANES = 16

GROUP = 128
NGROUP = HIST
NB = 4
PADW = 128
LAG = 2


def _emb_kernel(xt_hbm, null_hbm, oov_hbm, fixed_hbm, out_hbm,
                xbuf, sbuf, rowbuf, tbuf, mini, *sems):
    gsems = sems[:NB]
    osems = sems[NB:]
    wid = lax.axis_index("s") * NC + lax.axis_index("c")
    bbase = wid * GROUP

    pltpu.sync_copy(xt_hbm.at[:, pl.ds(bbase, GROUP)], xbuf)
    pltpu.sync_copy(null_hbm, mini.at[pl.ds(0, 1)])
    pltpu.sync_copy(oov_hbm, mini.at[pl.ds(1, 1)])

    rid0 = lax.iota(jnp.int32, LANES)

    def issue(g, b):
        for k in range(GROUP // LANES):
            xv = xbuf[g, pl.ds(k * LANES, LANES)]
            sbuf[g, pl.ds(k * LANES, LANES)] = jnp.maximum(xv - 2, 0) * 2
        pltpu.async_copy(fixed_hbm.at[sbuf.at[g]], rowbuf.at[b], gsems[b])

    def wait_gather(g, b):
        pltpu.make_async_copy(
            fixed_hbm.at[sbuf.at[g]], rowbuf.at[b], gsems[b]).wait()

    def fixup(g, b):
        acc = jnp.zeros((LANES,), jnp.bool_)
        for k in range(GROUP // LANES):
            acc = jnp.logical_or(acc, xbuf[g, pl.ds(k * LANES, LANES)] < 2)

        @pl.when(jnp.any(acc))
        def _fix_group():
            def sub_body(k, carry):
                xv = xbuf[g, pl.ds(k * LANES, LANES)]

                @pl.when(jnp.any(xv < 2))
                def _fix_sub():
                    m = xv < 2
                    p = jnp.minimum(xv, 1)
                    rids = rid0 + k * LANES

                    def col_body(c, carry2):
                        cs = jnp.full((LANES,), 0, jnp.int32) + c
                        patch = plsc.load_gather(mini, [p, cs])
                        plsc.store_scatter(
                            rowbuf.at[b], [rids, cs], patch, mask=m)
                        return carry2

                    lax.fori_loop(0, DIM, col_body, 0)

                return carry

            lax.fori_loop(0, GROUP // LANES, sub_body, 0)

    def transpose(b, b2):
        def dd_body(dd, carry):
            for r in range(8):
                ds_ = jnp.full((LANES,), 0, jnp.int32) + (dd * 8 + r)
                for k in range(GROUP // LANES):
                    v = plsc.load_gather(rowbuf.at[b], [rid0 + k * LANES, ds_])
                    tbuf[b2, dd, r, pl.ds(k * LANES, LANES)] = v
            return carry

        lax.fori_loop(0, DIM // 8, dd_body, 0)

    def start_out(g, b2):
        pltpu.async_copy(
            tbuf.at[b2], out_hbm.at[g, :, wid], osems[b2])

    def wait_out(g, b2):
        pltpu.make_async_copy(
            tbuf.at[b2], out_hbm.at[g, :, wid], osems[b2]).wait()

    def consume(g, b2):
        wait_gather(g, b2)
        fixup(g, b2)
        transpose(b2, b2)
        start_out(g, b2)

    def block_body(blk, carry):
        for b in range(NB):
            g = blk * NB + b

            issue(g, b)
            b2 = (b - LAG) % NB

            @pl.when(g >= LAG + NB)
            def _drain_out():
                wait_out(g - LAG - NB, b2)

            @pl.when(g >= LAG)
            def _consume():
                consume(g - LAG, b2)

        return carry

    lax.fori_loop(0, NGROUP // NB, block_body, 0)

    for g in range(NGROUP - LAG, NGROUP):
        b = g % NB
        wait_out(g - NB, b)
        consume(g, b)
    for g in range(NGROUP - NB, NGROUP):
        wait_out(g, g % NB)


@jax.jit
def kernel(x, null_emb, oov_weight, fixed_weight):
    xt = x.T
    fp = jnp.pad(fixed_weight, ((0, 2), (0, DIM)))
    mesh = plsc.VectorSubcoreMesh(core_axis_name="c", subcore_axis_name="s")
    out_t = pl.kernel(
        _emb_kernel,
        mesh=mesh,
        compiler_params=pltpu.CompilerParams(
            needs_layout_passes=False, use_tc_tiling_on_sc=False),
        out_type=jax.ShapeDtypeStruct((HIST, DIM // 8, NW, 8, GROUP), jnp.float32),
        scratch_types=[
            pltpu.VMEM((NGROUP, GROUP), jnp.int32),
            pltpu.VMEM((NGROUP, GROUP), jnp.int32),
            pltpu.VMEM((NB, GROUP, DIM), jnp.float32),
            pltpu.VMEM((NB, DIM // 8, 8, GROUP), jnp.float32),
            pltpu.VMEM((2, DIM), jnp.float32),
        ] + [pltpu.SemaphoreType.DMA] * (2 * NB),
    )(xt, null_emb, oov_weight, fp.reshape(2 * VOCAB, DIM))
    return (out_t.transpose(0, 1, 3, 2, 4)
            .reshape(HIST, DIM, BATCH).transpose(2, 0, 1))


# speedup vs baseline: 2.0691x; 2.0691x over previous
import jax
import jax.numpy as jnp
from jax import lax
from jax.experimental import pallas as pl
from jax.experimental.pallas import tpu as pltpu
from jax.experimental.pallas import tpu_sc as plsc

VOCAB = 1000000
DIM = 64
BATCH = 4096
HIST = 200

NC = 2
NS = 16
NW = NC * NS
LANES = 16

TOTAL = BATCH * HIST
PER_W = TOTAL // NW
GROUP = 128
NGROUP = PER_W // GROUP
NB = 8
PADW = 128
LAG = 4


def _emb_kernel(x_hbm, null_hbm, oov_hbm, fixed_hbm, out_hbm,
                xbuf, sbuf, rowbuf, mini, *sems):
    gsems = sems[:NB]
    osems = sems[NB:]
    wid = lax.axis_index("s") * NC + lax.axis_index("c")

    pltpu.sync_copy(x_hbm.at[wid], xbuf)
    pltpu.sync_copy(null_hbm, mini.at[pl.ds(0, 1)])
    pltpu.sync_copy(oov_hbm, mini.at[pl.ds(1, 1)])

    rid0 = lax.iota(jnp.int32, LANES)

    def issue(g, b):
        for k in range(GROUP // LANES):
            xv = xbuf[g, pl.ds(k * LANES, LANES)]
            sbuf[g, pl.ds(k * LANES, LANES)] = jnp.maximum(xv - 2, 0) * 2
        pltpu.async_copy(fixed_hbm.at[sbuf.at[g]], rowbuf.at[b], gsems[b])

    def wait_gather(g, b):
        pltpu.make_async_copy(
            fixed_hbm.at[sbuf.at[g]], rowbuf.at[b], gsems[b]).wait()

    def fixup(g, b):
        acc = jnp.zeros((LANES,), jnp.bool_)
        for k in range(GROUP // LANES):
            acc = jnp.logical_or(acc, xbuf[g, pl.ds(k * LANES, LANES)] < 2)

        @pl.when(jnp.any(acc))
        def _fix_group():
            def sub_body(k, carry):
                xv = xbuf[g, pl.ds(k * LANES, LANES)]

                @pl.when(jnp.any(xv < 2))
                def _fix_sub():
                    m = xv < 2
                    p = jnp.minimum(xv, 1)
                    rids = rid0 + k * LANES

                    def col_body(c, carry2):
                        cs = jnp.full((LANES,), 0, jnp.int32) + c
                        patch = plsc.load_gather(mini, [p, cs])
                        plsc.store_scatter(
                            rowbuf.at[b], [rids, cs], patch, mask=m)
                        return carry2

                    lax.fori_loop(0, DIM, col_body, 0)

                return carry

            lax.fori_loop(0, GROUP // LANES, sub_body, 0)

    def start_out(g, b):
        pltpu.async_copy(rowbuf.at[b], out_hbm.at[wid, g, :, pl.ds(0, DIM)], osems[b])

    def wait_out(g, b):
        pltpu.make_async_copy(
            rowbuf.at[b], out_hbm.at[wid, g, :, pl.ds(0, DIM)], osems[b]).wait()

    def block_body(blk, carry):
        for b in range(NB):
            g = blk * NB + b

            @pl.when(g >= NB)
            def _drain_out():
                wait_out(g - NB, b)

            issue(g, b)

            b2 = (b - LAG) % NB

            @pl.when(g >= LAG)
            def _consume():
                gm = g - LAG
                wait_gather(gm, b2)
                fixup(gm, b2)
                start_out(gm, b2)

        return carry

    lax.fori_loop(0, NGROUP // NB, block_body, 0)

    for g in range(NGROUP - LAG, NGROUP):
        b = g % NB
        wait_gather(g, b)
        fixup(g, b)
        start_out(g, b)
    for g in range(NGROUP - NB, NGROUP):
        wait_out(g, g % NB)


@jax.jit
def kernel(x, null_emb, oov_weight, fixed_weight):
    xg = x.reshape(NW, NGROUP, GROUP)
    fp = jnp.pad(fixed_weight, ((0, 2), (0, DIM)))
    mesh = plsc.VectorSubcoreMesh(core_axis_name="c", subcore_axis_name="s")
    out = pl.kernel(
        _emb_kernel,
        mesh=mesh,
        compiler_params=pltpu.CompilerParams(
            needs_layout_passes=False, use_tc_tiling_on_sc=False),
        out_type=jax.ShapeDtypeStruct((NW, NGROUP, GROUP, PADW), jnp.float32),
        scratch_types=[
            pltpu.VMEM((NGROUP, GROUP), jnp.int32),
            pltpu.VMEM((NGROUP, GROUP), jnp.int32),
            pltpu.VMEM((NB, GROUP, DIM), jnp.float32),
            pltpu.VMEM((2, DIM), jnp.float32),
        ] + [pltpu.SemaphoreType.DMA] * (2 * NB),
    )(xg, null_emb, oov_weight, fp.reshape(2 * VOCAB, DIM))
    return out[..., :DIM].reshape(BATCH, HIST, DIM)
